# Initial kernel scaffold; baseline (speedup 1.0000x reference)
import jax
import jax.numpy as jnp
from jax.experimental import pallas as pl


def kernel(logits, logprobs, finished):
    raise NotImplementedError("write your pallas kernel here")


# speedup vs baseline: 106.2461x; 106.2461x over previous
import jax
import jax.numpy as jnp
from jax.experimental import pallas as pl

_BS = 32
_NB = 8
_VOCAB = 100000
_PAD = 1
_NEG = -1e32
_FMIN = -3.0e38
_CW = 1024
_NCH = 97
_TAIL = _VOCAB - _NCH * _CW
_BIGI = 2 ** 30


def _finish(tv, ti, m, lse, prior, fin, val_ref, par_ref, tok_ref):
    pr_col = prior.reshape(_NB, 1)
    cand_v = ((tv - m) - lse) + pr_col

    finb = fin.reshape(_NB, 1) != 0
    slot = jax.lax.broadcasted_iota(jnp.int32, (_NB, _NB), 1)
    cand_v = jnp.where(finb, jnp.where(slot == 0, pr_col, _NEG), cand_v)
    cand_t = jnp.where(finb, jnp.where(slot == 0, _PAD, _VOCAB - 1), ti)

    beam = jax.lax.broadcasted_iota(jnp.int32, (_NB, _NB), 0)
    flat = beam * _VOCAB + cand_t
    out_v, out_f = [], []
    v = cand_v
    for _ in range(_NB):
        mj = jnp.max(v, axis=(0, 1), keepdims=True)
        fj = jnp.min(jnp.where(v == mj, flat, _BIGI), axis=(0, 1),
                     keepdims=True)
        out_v.append(mj)
        out_f.append(fj)
        v = jnp.where(flat == fj, _NEG, v)
    ov = jnp.concatenate(out_v, axis=1)
    of = jnp.concatenate(out_f, axis=1)
    parents = of // _VOCAB
    tokens = of - parents * _VOCAB

    val_ref[0] = ov
    par_ref[0] = parents
    tok_ref[0] = tokens


def _topk_step(logits_ref, prior_ref, fin_ref, val_ref, par_ref, tok_ref):
    x = logits_ref[0]
    prior = prior_ref[0]
    fin = fin_ref[0]

    m1 = jnp.full((_NB, _CW), _FMIN, dtype=jnp.float32)
    m2 = jnp.full((_NB, _CW), _FMIN, dtype=jnp.float32)
    c1 = jnp.zeros((_NB, _CW), dtype=jnp.int32)
    c2 = jnp.zeros((_NB, _CW), dtype=jnp.int32)
    for c in range(_NCH):
        xc = x[:, c * _CW:(c + 1) * _CW]
        g1 = xc > m1
        g2 = xc > m2
        m2 = jnp.where(g1, m1, jnp.where(g2, xc, m2))
        c2 = jnp.where(g1, c1, jnp.where(g2, c, c2))
        m1 = jnp.where(g1, xc, m1)
        c1 = jnp.where(g1, c, c1)
    xt = x[:, _NCH * _CW:]

    lane = jax.lax.broadcasted_iota(jnp.int32, (_NB, _CW), 1)
    vi1 = c1 * _CW + lane
    vi2 = c2 * _CW + lane
    vit = _NCH * _CW + jax.lax.broadcasted_iota(jnp.int32, (_NB, _TAIL), 1)

    m = jnp.maximum(jnp.max(m1, axis=1, keepdims=True),
                    jnp.max(xt, axis=1, keepdims=True))

    p1, p2, pt = m1, m2, xt
    top_v, top_i = [], []
    for _ in range(_NB):
        mj = jnp.maximum(
            jnp.maximum(jnp.max(p1, axis=1, keepdims=True),
                        jnp.max(p2, axis=1, keepdims=True)),
            jnp.max(pt, axis=1, keepdims=True))
        ij = jnp.minimum(
            jnp.minimum(
                jnp.min(jnp.where(p1 == mj, vi1, _BIGI), axis=1,
                        keepdims=True),
                jnp.min(jnp.where(p2 == mj, vi2, _BIGI), axis=1,
                        keepdims=True)),
            jnp.min(jnp.where(pt == mj, vit, _BIGI), axis=1,
                    keepdims=True))
        top_v.append(mj)
        top_i.append(ij)
        p1 = jnp.where(vi1 == ij, _FMIN, p1)
        p2 = jnp.where(vi2 == ij, _FMIN, p2)
        pt = jnp.where(vit == ij, _FMIN, pt)
    tv = jnp.concatenate(top_v, axis=1)
    ti = jnp.concatenate(top_i, axis=1)
    v8 = top_v[-1]

    ps = jnp.zeros((_NB, _CW), dtype=jnp.float32)
    cnt = jnp.zeros((_NB, _CW), dtype=jnp.int32)
    for c in range(_NCH):
        xc = x[:, c * _CW:(c + 1) * _CW]
        ps = ps + jnp.exp(xc - m)
        cnt = cnt + (xc >= v8).astype(jnp.int32)
    s = (jnp.sum(ps, axis=1, keepdims=True)
         + jnp.sum(jnp.exp(xt - m), axis=1, keepdims=True))
    lse = jnp.log(s)

    _finish(tv, ti, m, lse, prior, fin, val_ref, par_ref, tok_ref)

    bad = jnp.any(cnt >= 3)

    @pl.when(bad)
    def _fallback():
        iota = jax.lax.broadcasted_iota(jnp.int32, x.shape, 1)
        vals = x
        ftv, fti = [], []
        for _ in range(_NB):
            fm = jnp.max(vals, axis=1, keepdims=True)
            fi = jnp.min(jnp.where(vals == fm, iota, _BIGI), axis=1,
                         keepdims=True)
            ftv.append(fm)
            fti.append(fi)
            vals = jnp.where(iota == fi, _NEG, vals)
        _finish(jnp.concatenate(ftv, axis=1), jnp.concatenate(fti, axis=1),
                m, lse, prior, fin, val_ref, par_ref, tok_ref)


def kernel(logits, logprobs, finished):
    lg = logits.reshape(_BS, _NB, _VOCAB)
    pr = logprobs.reshape(_BS, 1, _NB)
    fin = finished.astype(jnp.int32).reshape(_BS, 1, _NB)

    out = pl.pallas_call(
        _topk_step,
        grid=(_BS,),
        in_specs=[
            pl.BlockSpec((1, _NB, _VOCAB), lambda i: (i, 0, 0)),
            pl.BlockSpec((1, 1, _NB), lambda i: (i, 0, 0)),
            pl.BlockSpec((1, 1, _NB), lambda i: (i, 0, 0)),
        ],
        out_specs=[
            pl.BlockSpec((1, 1, _NB), lambda i: (i, 0, 0)),
            pl.BlockSpec((1, 1, _NB), lambda i: (i, 0, 0)),
            pl.BlockSpec((1, 1, _NB), lambda i: (i, 0, 0)),
        ],
        out_shape=[
            jax.ShapeDtypeStruct((_BS, 1, _NB), jnp.float32),
            jax.ShapeDtypeStruct((_BS, 1, _NB), jnp.int32),
            jax.ShapeDtypeStruct((_BS, 1, _NB), jnp.int32),
        ],
    )(lg, pr, fin)
    tv, par, tok = out
    return (tv.reshape(1, _BS, _NB), par.reshape(1, _BS, _NB),
            tok.reshape(1, _BS, _NB))
